# Initial kernel scaffold; baseline (speedup 1.0000x reference)
import jax
import jax.numpy as jnp
from jax.experimental import pallas as pl


def kernel(x, edge_index, Wl1, Wr1, b1, Wl2, Wr2, b2):
    raise NotImplementedError("write your pallas kernel here")


# speedup vs baseline: 3.1047x; 3.1047x over previous
import functools

import jax
import jax.numpy as jnp
from jax import lax
from jax.experimental import pallas as pl
from jax.experimental.pallas import tpu as pltpu
from jax.experimental.pallas import tpu_sc as plsc

_N = 10000
_E = 160000
_NP = 10240
_NC = 2
_NS = 16
_K = 128
_CH = 80
_EPT = _CH * _K
_EP = _NS * _EPT
_RPT = _NP // _NS
_CR = _NP // 16


def _sc_agg_body(n_pass, with_counts, *refs):
    it = iter(refs)
    table = next(it)
    src_h = next(it)
    dst_h = next(it)
    zrow_h = next(it)
    if with_counts:
        zflat_h = next(it)
    agg_o = next(it)
    if with_counts:
        cnt_o = next(it)
    acc = next(it)
    idx_s = next(it)
    idx_d = next(it)
    rows = next(it)
    if with_counts:
        hist = next(it)
    gsem = next(it)

    c = lax.axis_index("c")
    t = lax.axis_index("s")

    pltpu.sync_copy(src_h.at[t], idx_s)
    pltpu.sync_copy(dst_h.at[t], idx_d)
    if with_counts:
        @pl.when(c == 0)
        def _():
            pltpu.sync_copy(zflat_h, hist)
            ones = jnp.full((16,), 1.0, jnp.float32)

            def cbody(k, carry):
                for i in range(_K // 16):
                    v = idx_d[k, pl.ds(i * 16, 16)]
                    plsc.addupdate_scatter(hist, [v], ones)
                return carry

            lax.fori_loop(0, _CH, cbody, 0)
            pltpu.sync_copy(hist, cnt_o.at[t])

    for p in range(n_pass):
        pltpu.sync_copy(zrow_h, acc.at[pl.ds(t * _RPT, _RPT)])
        plsc.subcore_barrier()

        chunk = 2 * p + c

        def ebody(k, carry):
            pltpu.async_copy(table.at[chunk].at[idx_s.at[k]], rows, gsem).wait()
            pltpu.sync_copy(rows, acc.at[idx_d.at[k]], add=True)
            return carry

        lax.fori_loop(0, _CH, ebody, 0)
        plsc.subcore_barrier()

        pltpu.sync_copy(acc.at[pl.ds(t * _RPT, _RPT)],
                        agg_o.at[chunk].at[pl.ds(t * _RPT, _RPT)])
        plsc.subcore_barrier()


@functools.lru_cache(maxsize=None)
def _make_sc_agg(n_pass, with_counts):
    f32 = jnp.float32
    out_type = [jax.ShapeDtypeStruct((n_pass * 2, _NP, 128), f32)]
    if with_counts:
        out_type.append(jax.ShapeDtypeStruct((_NS, _NP), f32))
    scratch = [
        pltpu.VMEM_SHARED((_NP, 128), f32),
        pltpu.VMEM((_CH, _K), jnp.int32),
        pltpu.VMEM((_CH, _K), jnp.int32),
        pltpu.VMEM((_K, 128), f32),
    ]
    if with_counts:
        scratch.append(pltpu.VMEM((_NP,), f32))
    scratch.append(pltpu.SemaphoreType.DMA)
    mesh = plsc.VectorSubcoreMesh(core_axis_name="c", subcore_axis_name="s",
                                  num_cores=_NC, num_subcores=_NS)
    return pl.kernel(
        functools.partial(_sc_agg_body, n_pass, with_counts),
        out_type=out_type,
        mesh=mesh,
        scratch_types=scratch,
        compiler_params=pltpu.CompilerParams(needs_layout_passes=False),
    )


def _make_tc1(bn):
    f32 = jnp.float32

    def body(agg_r, cnt16_r, x_r, wl_r, wr_r, b_r, out_r, cnt_o):
        a = jnp.concatenate([agg_r[0], agg_r[1]], axis=1)
        xx = jnp.concatenate([x_r[0], x_r[1]], axis=1)
        cnt = jnp.sum(cnt16_r[...], axis=0)[:, None]
        cnt_o[...] = cnt
        inv = 1.0 / jnp.maximum(cnt, 1.0)
        h = jnp.dot(a * inv, wl_r[...], preferred_element_type=f32)
        h = h + jnp.dot(xx, wr_r[...], preferred_element_type=f32)
        h = jnp.maximum(h + b_r[...], 0.0)
        for j in range(4):
            out_r[j] = h[:, j * 128:(j + 1) * 128]

    return pl.pallas_call(
        body,
        grid=(_NP // bn,),
        in_specs=[
            pl.BlockSpec((2, bn, 128), lambda i: (0, i, 0)),
            pl.BlockSpec((_NS, bn), lambda i: (0, i)),
            pl.BlockSpec((2, bn, 128), lambda i: (0, i, 0)),
            pl.BlockSpec((256, 512), lambda i: (0, 0)),
            pl.BlockSpec((256, 512), lambda i: (0, 0)),
            pl.BlockSpec((1, 512), lambda i: (0, 0)),
        ],
        out_specs=[
            pl.BlockSpec((4, bn, 128), lambda i: (0, i, 0)),
            pl.BlockSpec((bn, 1), lambda i: (i, 0)),
        ],
        out_shape=[
            jax.ShapeDtypeStruct((4, _NP, 128), f32),
            jax.ShapeDtypeStruct((_NP, 1), f32),
        ],
    )


def _make_tc2(bn):
    f32 = jnp.float32

    def body(agg_r, cnt_r, h_r, wl_r, wr_r, b_r, out_r):
        a = jnp.concatenate([agg_r[j] for j in range(4)], axis=1)
        hh = jnp.concatenate([h_r[j] for j in range(4)], axis=1)
        inv = 1.0 / jnp.maximum(cnt_r[...], 1.0)
        o = jnp.dot(a * inv, wl_r[...], preferred_element_type=f32)
        o = o + jnp.dot(hh, wr_r[...], preferred_element_type=f32)
        out_r[...] = o + b_r[...]

    return pl.pallas_call(
        body,
        grid=(_N // bn,),
        in_specs=[
            pl.BlockSpec((4, bn, 128), lambda i: (0, i, 0)),
            pl.BlockSpec((bn, 1), lambda i: (i, 0)),
            pl.BlockSpec((4, bn, 128), lambda i: (0, i, 0)),
            pl.BlockSpec((512, 512), lambda i: (0, 0)),
            pl.BlockSpec((512, 512), lambda i: (0, 0)),
            pl.BlockSpec((1, 512), lambda i: (0, 0)),
        ],
        out_specs=pl.BlockSpec((bn, 512), lambda i: (i, 0)),
        out_shape=jax.ShapeDtypeStruct((_N, 512), f32),
    )


def _sc_agg1(*args):
    return _make_sc_agg(1, True)(*args)


def _sc_agg2(*args):
    return _make_sc_agg(2, False)(*args)


_tc1 = _make_tc1(1024)
_tc2 = _make_tc2(400)


def kernel(x, edge_index, Wl1, Wr1, b1, Wl2, Wr2, b2):
    f32 = jnp.float32
    src = edge_index[0].astype(jnp.int32)
    dst = edge_index[1].astype(jnp.int32)
    pad = _EP - _E
    fill = jnp.full((pad,), _N, jnp.int32)
    src_r = jnp.concatenate([src, fill]).reshape(_NS, _CH, _K)
    dst_r = jnp.concatenate([dst, fill]).reshape(_NS, _CH, _K)

    xp = jnp.pad(x.astype(f32), ((0, _NP - _N), (0, 0)))
    x2 = xp.reshape(_NP, 2, 128).transpose(1, 0, 2)

    zrow = jnp.zeros((_RPT, 128), f32)
    zflat = jnp.zeros((_NP,), f32)

    agg1, cnt16 = _sc_agg1(x2, src_r, dst_r, zrow, zflat)
    h4, cntc = _tc1(agg1, cnt16, x2, Wl1.T, Wr1.T, b1.reshape(1, -1))
    (agg2,) = _sc_agg2(h4, src_r, dst_r, zrow)
    out = _tc2(agg2, cntc, h4, Wl2.T, Wr2.T, b2.reshape(1, -1))
    return out
